# baseline (speedup 1.0000x reference)
import functools

import jax
import jax.numpy as jnp
from jax import lax
from jax.experimental import pallas as pl
from jax.experimental.pallas import tpu as pltpu

N = 8192
D = 4096
NBH = 64
NB = 128
BR = 512
CB = 256


def _labels_body(x_ref, r_ref, lab_ref):
    x = x_ref[...]
    r = r_ref[...]
    xr = jnp.dot(x, r, preferred_element_type=jnp.float32)
    vmax = jnp.max(xr, axis=1, keepdims=True)
    vmin = jnp.min(xr, axis=1, keepdims=True)
    iota = lax.broadcasted_iota(jnp.int32, (BR, NBH), 1)
    amax = jnp.min(jnp.where(xr == vmax, iota, NBH), axis=1, keepdims=True)
    amin = jnp.min(jnp.where(xr == vmin, iota, NBH), axis=1, keepdims=True)
    lab_ref[...] = jnp.where(vmax >= -vmin, amax, NBH + amin)


def _pos_body(lab_ref, pos_ref):
    iota_b = lax.broadcasted_iota(jnp.int32, (CB, NB), 1)
    row_i = lax.broadcasted_iota(jnp.int32, (CB, CB), 0)
    col_j = lax.broadcasted_iota(jnp.int32, (CB, CB), 1)
    tri = (col_j < row_i).astype(jnp.float32)
    bU = (lax.broadcasted_iota(jnp.int32, (NB, NB), 0)
          < lax.broadcasted_iota(jnp.int32, (NB, NB), 1)).astype(jnp.float32)

    nk = N // CB

    def count_step(k, counts):
        lab = lab_ref[pl.ds(k * CB, CB), :]
        oh = (lab == iota_b).astype(jnp.float32)
        return counts + jnp.sum(oh, axis=0, keepdims=True)

    counts = lax.fori_loop(0, nk, count_step,
                           jnp.zeros((1, NB), jnp.float32))
    bucket_start = jnp.dot(counts, bU, preferred_element_type=jnp.float32)

    def pos_step(k, carry):
        lab = lab_ref[pl.ds(k * CB, CB), :]
        oh = (lab == iota_b).astype(jnp.float32)
        rank = jnp.dot(tri, oh, preferred_element_type=jnp.float32) + carry
        posv = jnp.sum(oh * (rank + bucket_start), axis=1, keepdims=True)
        pos_ref[pl.ds(k * CB, CB), :] = posv.astype(jnp.int32)
        return carry + jnp.sum(oh, axis=0, keepdims=True)

    lax.fori_loop(0, nk, pos_step, jnp.zeros((1, NB), jnp.float32))


def _labels_call(inp):
    return pl.pallas_call(
        _labels_body,
        grid=(N // BR,),
        in_specs=[
            pl.BlockSpec((BR, D), lambda i: (i, 0)),
            pl.BlockSpec((D, NBH), lambda i: (0, 0)),
        ],
        out_specs=pl.BlockSpec((BR, 1), lambda i: (i, 0)),
        out_shape=jax.ShapeDtypeStruct((N, 1), jnp.int32),
    )(*inp)


def _pos_call(labels):
    return pl.pallas_call(
        _pos_body,
        out_shape=jax.ShapeDtypeStruct((N, 1), jnp.int32),
    )(labels)


def kernel(input, R):
    labels = _labels_call((input, R))
    pos = _pos_call(labels)[:, 0]
    out = jnp.zeros_like(input).at[pos, :].set(input)
    return out


# speedup vs baseline: 2.0470x; 2.0470x over previous
import functools

import jax
import jax.numpy as jnp
from jax import lax
from jax.experimental import pallas as pl
from jax.experimental.pallas import tpu as pltpu
from jax.experimental.pallas import tpu_sc as plsc

N = 8192
D = 4096
NBH = 64
NB = 128
BR = 512
CB = 256


def _labels_body(x_ref, r_ref, lab_ref):
    x = x_ref[...]
    r = r_ref[...]
    xr = jnp.dot(x, r, preferred_element_type=jnp.float32)
    vmax = jnp.max(xr, axis=1, keepdims=True)
    vmin = jnp.min(xr, axis=1, keepdims=True)
    iota = lax.broadcasted_iota(jnp.int32, (BR, NBH), 1)
    amax = jnp.min(jnp.where(xr == vmax, iota, NBH), axis=1, keepdims=True)
    amin = jnp.min(jnp.where(xr == vmin, iota, NBH), axis=1, keepdims=True)
    lab_ref[...] = jnp.where(vmax >= -vmin, amax, NBH + amin)


def _pos_body(lab_ref, pos_ref):
    iota_b = lax.broadcasted_iota(jnp.int32, (CB, NB), 1)
    row_i = lax.broadcasted_iota(jnp.int32, (CB, CB), 0)
    col_j = lax.broadcasted_iota(jnp.int32, (CB, CB), 1)
    tri = (col_j < row_i).astype(jnp.float32)
    bU = (lax.broadcasted_iota(jnp.int32, (NB, NB), 0)
          < lax.broadcasted_iota(jnp.int32, (NB, NB), 1)).astype(jnp.float32)

    nk = N // CB

    def count_step(k, counts):
        lab = lab_ref[pl.ds(k * CB, CB), :]
        oh = (lab == iota_b).astype(jnp.float32)
        return counts + jnp.sum(oh, axis=0, keepdims=True)

    counts = lax.fori_loop(0, nk, count_step,
                           jnp.zeros((1, NB), jnp.float32))
    bucket_start = jnp.dot(counts, bU, preferred_element_type=jnp.float32)

    def pos_step(k, carry):
        lab = lab_ref[pl.ds(k * CB, CB), :]
        oh = (lab == iota_b).astype(jnp.float32)
        rank = jnp.dot(tri, oh, preferred_element_type=jnp.float32) + carry
        posv = jnp.sum(oh * (rank + bucket_start), axis=1, keepdims=True)
        pos_ref[pl.ds(k * CB, CB), :] = posv.astype(jnp.int32)
        return carry + jnp.sum(oh, axis=0, keepdims=True)

    lax.fori_loop(0, nk, pos_step, jnp.zeros((1, NB), jnp.float32))


def _labels_call(inp):
    return pl.pallas_call(
        _labels_body,
        grid=(N // BR,),
        in_specs=[
            pl.BlockSpec((BR, D), lambda i: (i, 0)),
            pl.BlockSpec((D, NBH), lambda i: (0, 0)),
        ],
        out_specs=pl.BlockSpec((BR, 1), lambda i: (i, 0)),
        out_shape=jax.ShapeDtypeStruct((N, 1), jnp.int32),
    )(*inp)


def _pos_call(labels):
    return pl.pallas_call(
        _pos_body,
        out_shape=jax.ShapeDtypeStruct((N, 1), jnp.int32),
    )(labels)


NW = 32
RPW = N // NW
G = 8
NCH = RPW // G


def _scatter_call(inp, pos3):
    mesh = plsc.VectorSubcoreMesh(core_axis_name="c", subcore_axis_name="s")

    @functools.partial(
        pl.kernel,
        mesh=mesh,
        out_type=jax.ShapeDtypeStruct((N, D), jnp.float32),
        scratch_types=[
            pltpu.VMEM((NCH, G), jnp.int32),
            pltpu.VMEM((G, D), jnp.float32),
            pltpu.VMEM((G, D), jnp.float32),
            pltpu.SemaphoreType.DMA,
            pltpu.SemaphoreType.DMA,
            pltpu.SemaphoreType.DMA,
        ],
    )
    def body(inp_hbm, pos_hbm, out_hbm, idx_v, buf0, buf1,
             sem_in, sem_o0, sem_o1):
        wid = lax.axis_index("s") * 2 + lax.axis_index("c")
        base = wid * RPW
        pltpu.sync_copy(pos_hbm.at[wid], idx_v)
        bufs = (buf0, buf1)
        osems = (sem_o0, sem_o1)

        def chunk(c, wait_prev):
            buf = bufs[c % 2]
            rd = pltpu.make_async_copy(
                inp_hbm.at[pl.ds(base + c * G, G)], buf, sem_in)
            rd.start()
            if wait_prev is not None:
                wait_prev()
            rd.wait()
            wr = pltpu.make_async_copy(buf, out_hbm.at[idx_v.at[c]],
                                       osems[c % 2])
            wr.start()
            return wr.wait

        wait_prev = None
        for c in range(NCH):
            wait_prev = chunk(c, wait_prev)
        wait_prev()

    return body(inp, pos3)


def kernel(input, R):
    labels = _labels_call((input, R))
    pos = _pos_call(labels)
    pos3 = pos.reshape(NW, NCH, G)
    return _scatter_call(input, pos3)
